# baseline (speedup 1.0000x reference)
import functools
import math

import jax
import jax.numpy as jnp
from jax import lax
from jax.experimental import pallas as pl
from jax.experimental.pallas import tpu as pltpu
from jax.experimental.pallas import tpu_sc as plsc

H = 64
HH = 32
NG = 50
CUTOFF = 5.0
NI = 3
NZ = 100
N = 50000
E = 800000

NC = 2
NS = 16
LN = 16

EPW = 25088
E_D = NC * NS * EPW
DW = 512
NWIN = EPW // DW


def _sc_dist_body(px_ref, py_ref, pz_ref, i_ref, j_ref, out_ref, coord, acc, ib, jb):
    c = lax.axis_index("c")
    s = lax.axis_index("s")
    base = (s * NC + c) * EPW

    for p, cref in enumerate((px_ref, py_ref, pz_ref)):
        pltpu.sync_copy(cref, coord)

        def win(w, _, p=p):
            e0 = base + w * DW
            pltpu.sync_copy(i_ref.at[pl.ds(e0, DW)], ib)
            pltpu.sync_copy(j_ref.at[pl.ds(e0, DW)], jb)
            for k in range(DW // LN):
                sl = pl.ds(w * DW + k * LN, LN)
                iv = ib[pl.ds(k * LN, LN)]
                jv = jb[pl.ds(k * LN, LN)]
                dfr = plsc.load_gather(coord, [jv]) - plsc.load_gather(coord, [iv])
                sq = dfr * dfr
                if p == 0:
                    acc[sl] = sq
                else:
                    acc[sl] = acc[sl] + sq
            return 0

        lax.fori_loop(0, NWIN, win, 0)
    pltpu.sync_copy(acc, out_ref.at[pl.ds(base, EPW)])


def _sc_dist(px, py, pz, i_d, j_d):
    mesh = plsc.VectorSubcoreMesh(core_axis_name="c", subcore_axis_name="s")
    return pl.kernel(
        _sc_dist_body,
        out_type=jax.ShapeDtypeStruct((E_D,), jnp.float32),
        mesh=mesh,
        scratch_types=[
            pltpu.VMEM((N,), jnp.float32),
            pltpu.VMEM((EPW,), jnp.float32),
            pltpu.VMEM((DW,), jnp.int32),
            pltpu.VMEM((DW,), jnp.int32),
        ],
        compiler_params=pltpu.CompilerParams(needs_layout_passes=False),
    )(px, py, pz, i_d, j_d)


EPT = E // NS
GW = 80
SUP = 400
WPS = SUP // GW
NSUP = EPT // SUP
NP = 50048
RPT = NP // NS
ZR = 184


def _sc_edge_body(t, ytab_ref, wf_ref, iw_ref, jw_ref, out_ref,
                  acc, ibl, jbl, ib2, jadj, wfb, rows, sem):
    c = lax.axis_index("c")
    s = lax.axis_index("s")
    cN = c * N

    zero = jnp.zeros((LN,), jnp.float32)
    for r in range(ZR):
        rows[r, pl.ds(0, LN)] = zero
        rows[r, pl.ds(LN, LN)] = zero
    for rb in range(RPT // ZR):
        pltpu.sync_copy(rows.at[pl.ds(0, ZR)],
                        acc.at[pl.ds(s * RPT + rb * ZR, ZR)])
    plsc.subcore_barrier()

    def sup(sw, _):
        e0 = s * EPT + sw * SUP
        pltpu.sync_copy(iw_ref.at[pl.ds(e0, SUP)], ibl)
        pltpu.sync_copy(jw_ref.at[pl.ds(e0, SUP)], jbl)
        pltpu.sync_copy(wf_ref.at[t, c, pl.ds(e0, SUP)], wfb)
        for w in range(WPS):
            for q in range(GW // LN):
                f0 = w * GW + q * LN
                ib2[w, pl.ds(q * LN, LN)] = ibl[pl.ds(f0, LN)]
                jadj[w, pl.ds(q * LN, LN)] = jbl[pl.ds(f0, LN)] + cN
        cps = [
            pltpu.async_copy(ytab_ref.at[jadj.at[w]],
                             rows.at[pl.ds(w * GW, GW)], sem)
            for w in range(WPS)
        ]
        for cp in cps:
            cp.wait()

        def mul(rr, _):
            for dr in range(8):
                ri = rr * 8 + dr
                for h0 in (0, LN):
                    rows[ri, pl.ds(h0, LN)] = (
                        rows[ri, pl.ds(h0, LN)] * wfb[ri, pl.ds(h0, LN)])
            return 0

        lax.fori_loop(0, SUP // 8, mul, 0)
        for w in range(WPS):
            pltpu.sync_copy(rows.at[pl.ds(w * GW, GW)], acc.at[ib2.at[w]],
                            add=True)
        return 0

    lax.fori_loop(0, NSUP, sup, 0)
    plsc.subcore_barrier()
    pltpu.sync_copy(acc.at[pl.ds(s * RPT, RPT)],
                    out_ref.at[c, pl.ds(s * RPT, RPT)])


def _sc_edge(t, ytab, wf, iw, jw):
    mesh = plsc.VectorSubcoreMesh(core_axis_name="c", subcore_axis_name="s")
    return pl.kernel(
        functools.partial(_sc_edge_body, t),
        out_type=jax.ShapeDtypeStruct((NC, NP, HH), jnp.float32),
        mesh=mesh,
        scratch_types=[
            pltpu.VMEM_SHARED((NP, HH), jnp.float32),
            pltpu.VMEM((SUP,), jnp.int32),
            pltpu.VMEM((SUP,), jnp.int32),
            pltpu.VMEM((WPS, GW), jnp.int32),
            pltpu.VMEM((WPS, GW), jnp.int32),
            pltpu.VMEM((SUP, HH), jnp.float32),
            pltpu.VMEM((SUP, HH), jnp.float32),
            pltpu.SemaphoreType.DMA,
        ],
        compiler_params=pltpu.CompilerParams(needs_layout_passes=False,
                                             use_tc_tiling_on_sc=False),
    )(ytab, wf, iw, jw)


def _ssp(x):
    return jax.nn.softplus(x) - math.log(2.0)


BE = 3200


def _tc_filter_body(d2_ref, fw1_ref, fb1_ref, fw2_ref, fb2_ref, out_ref):
    d2 = d2_ref[0, 0, :]
    d = jnp.sqrt(d2 + 1e-12)
    width = CUTOFF / (NG - 1)
    offs = lax.broadcasted_iota(jnp.int32, (1, NG), 1).astype(jnp.float32) * width
    delta = d[:, None] - offs
    rbf = jnp.exp((-0.5 / (width * width)) * delta * delta)
    fcut = 0.5 * (jnp.cos(d * (math.pi / CUTOFF)) + 1.0)
    fcut = fcut * (d < CUTOFF).astype(jnp.float32)
    rbf16 = rbf.astype(jnp.bfloat16)
    for t in range(NI):
        h1 = _ssp(jnp.dot(rbf16, fw1_ref[t].astype(jnp.bfloat16),
                          preferred_element_type=jnp.float32)
                  + fb1_ref[t])
        wfv = jnp.dot(h1.astype(jnp.bfloat16), fw2_ref[t].astype(jnp.bfloat16),
                      preferred_element_type=jnp.float32) + fb2_ref[t]
        wfv = wfv * fcut[:, None]
        out_ref[t, 0] = wfv[:, :HH]
        out_ref[t, 1] = wfv[:, HH:]


def _tc_filter(d2, fw1, fb1, fw2, fb2):
    return pl.pallas_call(
        _tc_filter_body,
        grid=(E // BE,),
        in_specs=[
            pl.BlockSpec((1, 1, BE), lambda n: (n, 0, 0)),
            pl.BlockSpec((NI, NG, H), lambda n: (0, 0, 0)),
            pl.BlockSpec((NI, H), lambda n: (0, 0)),
            pl.BlockSpec((NI, H, H), lambda n: (0, 0, 0)),
            pl.BlockSpec((NI, H), lambda n: (0, 0)),
        ],
        out_specs=pl.BlockSpec((NI, 2, BE, HH), lambda n: (0, 0, n, 0)),
        out_shape=jax.ShapeDtypeStruct((NI, 2, E, HH), jnp.float32),
    )(d2, fw1, fb1, fw2, fb2)


BN = 2000


def _tc_node0_body(z_ref, emb_ref, inw_ref, x_ref, y2_ref):
    z = z_ref[0, 0, :]
    zi = lax.broadcasted_iota(jnp.int32, (BN, NZ), 1)
    oh = (z[:, None] == zi).astype(jnp.float32)
    x = jnp.dot(oh, emb_ref[...], preferred_element_type=jnp.float32)
    y = jnp.dot(x, inw_ref[...], preferred_element_type=jnp.float32)
    x_ref[...] = x
    y2_ref[0] = y[:, :HH]
    y2_ref[1] = y[:, HH:]


def _tc_node0(z, emb, inw):
    return pl.pallas_call(
        _tc_node0_body,
        grid=(N // BN,),
        in_specs=[
            pl.BlockSpec((1, 1, BN), lambda n: (n, 0, 0)),
            pl.BlockSpec((NZ, H), lambda n: (0, 0)),
            pl.BlockSpec((H, H), lambda n: (0, 0)),
        ],
        out_specs=[
            pl.BlockSpec((BN, H), lambda n: (n, 0)),
            pl.BlockSpec((2, BN, HH), lambda n: (0, n, 0)),
        ],
        out_shape=[
            jax.ShapeDtypeStruct((N, H), jnp.float32),
            jax.ShapeDtypeStruct((2, N, HH), jnp.float32),
        ],
    )(z, emb, inw)


def _node_update(agg2_ref, x_ref, f2w_ref, f2b_ref, ow_ref, ob_ref):
    pre = (jnp.dot(agg2_ref[0], f2w_ref[:HH, :], preferred_element_type=jnp.float32)
           + jnp.dot(agg2_ref[1], f2w_ref[HH:, :], preferred_element_type=jnp.float32)
           + f2b_ref[...])
    v = jnp.dot(_ssp(pre), ow_ref[...], preferred_element_type=jnp.float32) + ob_ref[...]
    return x_ref[...] + v


def _tc_node_body(agg2_ref, x_ref, f2w_ref, f2b_ref, ow_ref, ob_ref, inw_ref,
                  xn_ref, y2_ref):
    xn = _node_update(agg2_ref, x_ref, f2w_ref, f2b_ref, ow_ref, ob_ref)
    xn_ref[...] = xn
    y = jnp.dot(xn, inw_ref[...], preferred_element_type=jnp.float32)
    y2_ref[0] = y[:, :HH]
    y2_ref[1] = y[:, HH:]


def _tc_node(agg2, x, f2w, f2b, ow, ob, inw):
    return pl.pallas_call(
        _tc_node_body,
        grid=(N // BN,),
        in_specs=[
            pl.BlockSpec((2, BN, HH), lambda n: (0, n, 0)),
            pl.BlockSpec((BN, H), lambda n: (n, 0)),
            pl.BlockSpec((H, H), lambda n: (0, 0)),
            pl.BlockSpec((H,), lambda n: (0,)),
            pl.BlockSpec((H, H), lambda n: (0, 0)),
            pl.BlockSpec((H,), lambda n: (0,)),
            pl.BlockSpec((H, H), lambda n: (0, 0)),
        ],
        out_specs=[
            pl.BlockSpec((BN, H), lambda n: (n, 0)),
            pl.BlockSpec((2, BN, HH), lambda n: (0, n, 0)),
        ],
        out_shape=[
            jax.ShapeDtypeStruct((N, H), jnp.float32),
            jax.ShapeDtypeStruct((2, N, HH), jnp.float32),
        ],
    )(agg2, x, f2w, f2b, ow, ob, inw)


def _tc_node2_body(agg2_ref, x_ref, f2w_ref, f2b_ref, ow_ref, ob_ref,
                   w1_ref, b1_ref, w2_ref, b2_ref, out_ref):
    xn = _node_update(agg2_ref, x_ref, f2w_ref, f2b_ref, ow_ref, ob_ref)
    h = _ssp(jnp.dot(xn, w1_ref[...], preferred_element_type=jnp.float32) + b1_ref[...])
    out_ref[...] = jnp.dot(h, w2_ref[...], preferred_element_type=jnp.float32) + b2_ref[...]


def _tc_node2(agg2, x, f2w, f2b, ow, ob, w1, b1, w2, b2):
    return pl.pallas_call(
        _tc_node2_body,
        grid=(N // BN,),
        in_specs=[
            pl.BlockSpec((2, BN, HH), lambda n: (0, n, 0)),
            pl.BlockSpec((BN, H), lambda n: (n, 0)),
            pl.BlockSpec((H, H), lambda n: (0, 0)),
            pl.BlockSpec((H,), lambda n: (0,)),
            pl.BlockSpec((H, H), lambda n: (0, 0)),
            pl.BlockSpec((H,), lambda n: (0,)),
            pl.BlockSpec((H, H // 2), lambda n: (0, 0)),
            pl.BlockSpec((H // 2,), lambda n: (0,)),
            pl.BlockSpec((H // 2, 3), lambda n: (0, 0)),
            pl.BlockSpec((3,), lambda n: (0,)),
        ],
        out_specs=pl.BlockSpec((BN, 3), lambda n: (n, 0)),
        out_shape=jax.ShapeDtypeStruct((N, 3), jnp.float32),
    )(agg2, x, f2w, f2b, ow, ob, w1, b1, w2, b2)


def kernel(z, pos, edge_index, batch, params):
    i = edge_index[0].astype(jnp.int32)
    j = edge_index[1].astype(jnp.int32)
    pad = E_D - E
    i_d = jnp.concatenate([i, jnp.zeros((pad,), jnp.int32)])
    j_d = jnp.concatenate([j, jnp.zeros((pad,), jnp.int32)])
    d2 = _sc_dist(pos[:, 0], pos[:, 1], pos[:, 2], i_d, j_d)[:E]

    inter = params['interactions']
    fw1 = jnp.stack([p['fw1'] for p in inter])
    fb1 = jnp.stack([p['fb1'] for p in inter])
    fw2 = jnp.stack([p['fw2'] for p in inter])
    fb2 = jnp.stack([p['fb2'] for p in inter])
    wf = _tc_filter(d2.reshape(E // BE, 1, BE), fw1, fb1, fw2, fb2)

    x, y2 = _tc_node0(z.astype(jnp.int32).reshape(N // BN, 1, BN),
                      params['emb'], inter[0]['inw'])
    score = None
    for t in range(NI):
        ytab = y2.reshape(NC * N, HH)
        agg2 = _sc_edge(t, ytab, wf, i, j)
        p = inter[t]
        if t < NI - 1:
            x, y2 = _tc_node(agg2, x, p['f2w'], p['f2b'], p['ow'], p['ob'],
                             inter[t + 1]['inw'])
        else:
            score = _tc_node2(agg2, x, p['f2w'], p['f2b'], p['ow'], p['ob'],
                              params['out_w1'], params['out_b1'],
                              params['out_w2'], params['out_b2'])
    return score


# speedup vs baseline: 1.2139x; 1.2139x over previous
import functools
import math

import jax
import jax.numpy as jnp
from jax import lax
from jax.experimental import pallas as pl
from jax.experimental.pallas import tpu as pltpu
from jax.experimental.pallas import tpu_sc as plsc

H = 64
HH = 32
NG = 50
CUTOFF = 5.0
NI = 3
NZ = 100
N = 50000
E = 800000

NC = 2
NS = 16
LN = 16

EPW = 25088
E_D = NC * NS * EPW
DW = 512
NWIN = EPW // DW


def _sc_dist_body(px_ref, py_ref, pz_ref, i_ref, j_ref, out_ref, coord, acc, ib, jb):
    c = lax.axis_index("c")
    s = lax.axis_index("s")
    base = (s * NC + c) * EPW

    for p, cref in enumerate((px_ref, py_ref, pz_ref)):
        pltpu.sync_copy(cref, coord)

        def win(w, _, p=p):
            e0 = base + w * DW
            pltpu.sync_copy(i_ref.at[pl.ds(e0, DW)], ib)
            pltpu.sync_copy(j_ref.at[pl.ds(e0, DW)], jb)
            for k in range(DW // LN):
                sl = pl.ds(w * DW + k * LN, LN)
                iv = ib[pl.ds(k * LN, LN)]
                jv = jb[pl.ds(k * LN, LN)]
                dfr = plsc.load_gather(coord, [jv]) - plsc.load_gather(coord, [iv])
                sq = dfr * dfr
                if p == 0:
                    acc[sl] = sq
                else:
                    acc[sl] = acc[sl] + sq
            return 0

        lax.fori_loop(0, NWIN, win, 0)
    pltpu.sync_copy(acc, out_ref.at[pl.ds(base, EPW)])


def _sc_dist(px, py, pz, i_d, j_d):
    mesh = plsc.VectorSubcoreMesh(core_axis_name="c", subcore_axis_name="s")
    return pl.kernel(
        _sc_dist_body,
        out_type=jax.ShapeDtypeStruct((E_D,), jnp.float32),
        mesh=mesh,
        scratch_types=[
            pltpu.VMEM((N,), jnp.float32),
            pltpu.VMEM((EPW,), jnp.float32),
            pltpu.VMEM((DW,), jnp.int32),
            pltpu.VMEM((DW,), jnp.int32),
        ],
        compiler_params=pltpu.CompilerParams(needs_layout_passes=False),
    )(px, py, pz, i_d, j_d)


EPT = E_D // NS
GW = 128
SUP = 256
WPS = SUP // GW
NSUP = EPT // SUP
WFR = E_D // 4
NP = 50048
RPT = NP // NS
ZR = 184


def _sc_edge_body(t, ytab_ref, wf_ref, iw_ref, jw_ref, out_ref,
                  acc, ibl, jbl, ib2, jadj, wfb, rows, sem):
    c = lax.axis_index("c")
    s = lax.axis_index("s")
    cN = c * N

    zero = jnp.zeros((LN,), jnp.float32)
    for r in range(ZR):
        rows[r, pl.ds(0, LN)] = zero
        rows[r, pl.ds(LN, LN)] = zero
    for rb in range(RPT // ZR):
        pltpu.sync_copy(rows.at[pl.ds(0, ZR)],
                        acc.at[pl.ds(s * RPT + rb * ZR, ZR)])
    plsc.subcore_barrier()

    def sup(sw, _):
        e0 = s * EPT + sw * SUP
        pltpu.sync_copy(iw_ref.at[pl.ds(e0, SUP)], ibl)
        pltpu.sync_copy(jw_ref.at[pl.ds(e0, SUP)], jbl)
        pltpu.sync_copy(wf_ref.at[t, c, pl.ds(e0 // 4, SUP // 4)], wfb)
        for w in range(WPS):
            for q in range(GW // LN):
                f0 = w * GW + q * LN
                ib2[w, pl.ds(q * LN, LN)] = ibl[pl.ds(f0, LN)]
                jadj[w, pl.ds(q * LN, LN)] = jbl[pl.ds(f0, LN)] + cN
        cps = [
            pltpu.async_copy(ytab_ref.at[jadj.at[w]],
                             rows.at[pl.ds(w * GW, GW)], sem)
            for w in range(WPS)
        ]
        for cp in cps:
            cp.wait()

        def mul(q, _):
            for k in range(4):
                for h0 in (0, LN):
                    rows[q * 4 + k, pl.ds(h0, LN)] = (
                        rows[q * 4 + k, pl.ds(h0, LN)]
                        * wfb[q, pl.ds(k * HH + h0, LN)])
            return 0

        lax.fori_loop(0, SUP // 4, mul, 0)
        for w in range(WPS):
            pltpu.sync_copy(rows.at[pl.ds(w * GW, GW)], acc.at[ib2.at[w]],
                            add=True)
        return 0

    lax.fori_loop(0, NSUP, sup, 0)
    plsc.subcore_barrier()
    pltpu.sync_copy(acc.at[pl.ds(s * RPT, RPT)],
                    out_ref.at[c, pl.ds(s * RPT, RPT)])


def _sc_edge(t, ytab, wf, iw, jw):
    mesh = plsc.VectorSubcoreMesh(core_axis_name="c", subcore_axis_name="s")
    return pl.kernel(
        functools.partial(_sc_edge_body, t),
        out_type=jax.ShapeDtypeStruct((NC, NP, HH), jnp.float32),
        mesh=mesh,
        scratch_types=[
            pltpu.VMEM_SHARED((NP, HH), jnp.float32),
            pltpu.VMEM((SUP,), jnp.int32),
            pltpu.VMEM((SUP,), jnp.int32),
            pltpu.VMEM((WPS, GW), jnp.int32),
            pltpu.VMEM((WPS, GW), jnp.int32),
            pltpu.VMEM((SUP // 4, 128), jnp.float32),
            pltpu.VMEM((SUP, HH), jnp.float32),
            pltpu.SemaphoreType.DMA,
        ],
        compiler_params=pltpu.CompilerParams(needs_layout_passes=False,
                                             use_tc_tiling_on_sc=False),
    )(ytab, wf, iw, jw)


def _ssp(x):
    return jax.nn.softplus(x) - math.log(2.0)


BE = 3584


def _tc_filter_body(d2_ref, fw1_ref, fb1_ref, fw2_ref, fb2_ref, out_ref):
    d2 = d2_ref[0, 0, :]
    d = jnp.sqrt(d2 + 1e-12)
    width = CUTOFF / (NG - 1)
    offs = lax.broadcasted_iota(jnp.int32, (1, NG), 1).astype(jnp.float32) * width
    delta = d[:, None] - offs
    rbf = jnp.exp((-0.5 / (width * width)) * delta * delta)
    fcut = 0.5 * (jnp.cos(d * (math.pi / CUTOFF)) + 1.0)
    fcut = fcut * (d < CUTOFF).astype(jnp.float32)
    rbf16 = rbf.astype(jnp.bfloat16)
    for t in range(NI):
        h1 = _ssp(jnp.dot(rbf16, fw1_ref[t].astype(jnp.bfloat16),
                          preferred_element_type=jnp.float32)
                  + fb1_ref[t])
        wfv = jnp.dot(h1.astype(jnp.bfloat16), fw2_ref[t].astype(jnp.bfloat16),
                      preferred_element_type=jnp.float32) + fb2_ref[t]
        wfv = wfv * fcut[:, None]
        BQ = BE // 4
        for h, half in ((0, wfv[:, :HH]), (1, wfv[:, HH:])):
            out_ref[t, h] = jnp.concatenate(
                [half[k * BQ:(k + 1) * BQ] for k in range(4)], axis=1)


def _tc_filter(d2, fw1, fb1, fw2, fb2):
    return pl.pallas_call(
        _tc_filter_body,
        grid=(E_D // BE,),
        in_specs=[
            pl.BlockSpec((1, 1, BE), lambda n: (n, 0, 0)),
            pl.BlockSpec((NI, NG, H), lambda n: (0, 0, 0)),
            pl.BlockSpec((NI, H), lambda n: (0, 0)),
            pl.BlockSpec((NI, H, H), lambda n: (0, 0, 0)),
            pl.BlockSpec((NI, H), lambda n: (0, 0)),
        ],
        out_specs=pl.BlockSpec((NI, 2, BE // 4, 128), lambda n: (0, 0, n, 0)),
        out_shape=jax.ShapeDtypeStruct((NI, 2, WFR, 128), jnp.float32),
    )(d2, fw1, fb1, fw2, fb2)


BN = 2000


def _tc_node0_body(z_ref, emb_ref, inw_ref, x_ref, y2_ref):
    z = z_ref[0, 0, :]
    zi = lax.broadcasted_iota(jnp.int32, (BN, NZ), 1)
    oh = (z[:, None] == zi).astype(jnp.float32)
    x = jnp.dot(oh, emb_ref[...], preferred_element_type=jnp.float32)
    y = jnp.dot(x, inw_ref[...], preferred_element_type=jnp.float32)
    x_ref[...] = x
    y2_ref[0] = y[:, :HH]
    y2_ref[1] = y[:, HH:]


def _tc_node0(z, emb, inw):
    return pl.pallas_call(
        _tc_node0_body,
        grid=(N // BN,),
        in_specs=[
            pl.BlockSpec((1, 1, BN), lambda n: (n, 0, 0)),
            pl.BlockSpec((NZ, H), lambda n: (0, 0)),
            pl.BlockSpec((H, H), lambda n: (0, 0)),
        ],
        out_specs=[
            pl.BlockSpec((BN, H), lambda n: (n, 0)),
            pl.BlockSpec((2, BN, HH), lambda n: (0, n, 0)),
        ],
        out_shape=[
            jax.ShapeDtypeStruct((N, H), jnp.float32),
            jax.ShapeDtypeStruct((2, N, HH), jnp.float32),
        ],
    )(z, emb, inw)


def _node_update(agg2_ref, x_ref, f2w_ref, f2b_ref, ow_ref, ob_ref):
    pre = (jnp.dot(agg2_ref[0], f2w_ref[:HH, :], preferred_element_type=jnp.float32)
           + jnp.dot(agg2_ref[1], f2w_ref[HH:, :], preferred_element_type=jnp.float32)
           + f2b_ref[...])
    v = jnp.dot(_ssp(pre), ow_ref[...], preferred_element_type=jnp.float32) + ob_ref[...]
    return x_ref[...] + v


def _tc_node_body(agg2_ref, x_ref, f2w_ref, f2b_ref, ow_ref, ob_ref, inw_ref,
                  xn_ref, y2_ref):
    xn = _node_update(agg2_ref, x_ref, f2w_ref, f2b_ref, ow_ref, ob_ref)
    xn_ref[...] = xn
    y = jnp.dot(xn, inw_ref[...], preferred_element_type=jnp.float32)
    y2_ref[0] = y[:, :HH]
    y2_ref[1] = y[:, HH:]


def _tc_node(agg2, x, f2w, f2b, ow, ob, inw):
    return pl.pallas_call(
        _tc_node_body,
        grid=(N // BN,),
        in_specs=[
            pl.BlockSpec((2, BN, HH), lambda n: (0, n, 0)),
            pl.BlockSpec((BN, H), lambda n: (n, 0)),
            pl.BlockSpec((H, H), lambda n: (0, 0)),
            pl.BlockSpec((H,), lambda n: (0,)),
            pl.BlockSpec((H, H), lambda n: (0, 0)),
            pl.BlockSpec((H,), lambda n: (0,)),
            pl.BlockSpec((H, H), lambda n: (0, 0)),
        ],
        out_specs=[
            pl.BlockSpec((BN, H), lambda n: (n, 0)),
            pl.BlockSpec((2, BN, HH), lambda n: (0, n, 0)),
        ],
        out_shape=[
            jax.ShapeDtypeStruct((N, H), jnp.float32),
            jax.ShapeDtypeStruct((2, N, HH), jnp.float32),
        ],
    )(agg2, x, f2w, f2b, ow, ob, inw)


def _tc_node2_body(agg2_ref, x_ref, f2w_ref, f2b_ref, ow_ref, ob_ref,
                   w1_ref, b1_ref, w2_ref, b2_ref, out_ref):
    xn = _node_update(agg2_ref, x_ref, f2w_ref, f2b_ref, ow_ref, ob_ref)
    h = _ssp(jnp.dot(xn, w1_ref[...], preferred_element_type=jnp.float32) + b1_ref[...])
    out_ref[...] = jnp.dot(h, w2_ref[...], preferred_element_type=jnp.float32) + b2_ref[...]


def _tc_node2(agg2, x, f2w, f2b, ow, ob, w1, b1, w2, b2):
    return pl.pallas_call(
        _tc_node2_body,
        grid=(N // BN,),
        in_specs=[
            pl.BlockSpec((2, BN, HH), lambda n: (0, n, 0)),
            pl.BlockSpec((BN, H), lambda n: (n, 0)),
            pl.BlockSpec((H, H), lambda n: (0, 0)),
            pl.BlockSpec((H,), lambda n: (0,)),
            pl.BlockSpec((H, H), lambda n: (0, 0)),
            pl.BlockSpec((H,), lambda n: (0,)),
            pl.BlockSpec((H, H // 2), lambda n: (0, 0)),
            pl.BlockSpec((H // 2,), lambda n: (0,)),
            pl.BlockSpec((H // 2, 3), lambda n: (0, 0)),
            pl.BlockSpec((3,), lambda n: (0,)),
        ],
        out_specs=pl.BlockSpec((BN, 3), lambda n: (n, 0)),
        out_shape=jax.ShapeDtypeStruct((N, 3), jnp.float32),
    )(agg2, x, f2w, f2b, ow, ob, w1, b1, w2, b2)


def kernel(z, pos, edge_index, batch, params):
    i = edge_index[0].astype(jnp.int32)
    j = edge_index[1].astype(jnp.int32)
    pad = E_D - E
    i_d = jnp.concatenate([i, jnp.zeros((pad,), jnp.int32)])
    j_d = jnp.concatenate([j, jnp.zeros((pad,), jnp.int32)])
    d2 = _sc_dist(pos[:, 0], pos[:, 1], pos[:, 2], i_d, j_d)
    d2 = d2.at[E:].set(1e9)

    inter = params['interactions']
    fw1 = jnp.stack([p['fw1'] for p in inter])
    fb1 = jnp.stack([p['fb1'] for p in inter])
    fw2 = jnp.stack([p['fw2'] for p in inter])
    fb2 = jnp.stack([p['fb2'] for p in inter])
    wf = _tc_filter(d2.reshape(E_D // BE, 1, BE), fw1, fb1, fw2, fb2)

    i_p = i_d.reshape(E_D // BE, 4, BE // 4).transpose(0, 2, 1).reshape(E_D)
    j_p = j_d.reshape(E_D // BE, 4, BE // 4).transpose(0, 2, 1).reshape(E_D)

    x, y2 = _tc_node0(z.astype(jnp.int32).reshape(N // BN, 1, BN),
                      params['emb'], inter[0]['inw'])
    score = None
    for t in range(NI):
        ytab = y2.reshape(NC * N, HH)
        agg2 = _sc_edge(t, ytab, wf, i_p, j_p)
        p = inter[t]
        if t < NI - 1:
            x, y2 = _tc_node(agg2, x, p['f2w'], p['f2b'], p['ow'], p['ob'],
                             inter[t + 1]['inw'])
        else:
            score = _tc_node2(agg2, x, p['f2w'], p['f2b'], p['ow'], p['ob'],
                              params['out_w1'], params['out_b1'],
                              params['out_w2'], params['out_b2'])
    return score
